# Initial kernel scaffold; baseline (speedup 1.0000x reference)
import jax
import jax.numpy as jnp
from jax.experimental import pallas as pl


def kernel(input, W):
    raise NotImplementedError("write your pallas kernel here")


# speedup vs baseline: 1.2002x; 1.2002x over previous
import functools

import jax
import jax.numpy as jnp
from jax.experimental import pallas as pl
from jax.experimental.pallas import tpu as pltpu

_NUM_EXPERTS = 64
_TOP_K = 8
_TILE = 512


def _router_body(x_ref, w_ref, w_out_ref, idx_out_ref):
    x = x_ref[...]
    w = w_ref[...]
    logits = jax.lax.dot_general(
        x, w, (((1,), (1,)), ((), ())),
        preferred_element_type=jnp.float32)
    t = logits.shape[0]
    lane = jax.lax.broadcasted_iota(jnp.int32, logits.shape, 1)
    cols = jax.lax.broadcasted_iota(jnp.int32, (t, _TOP_K), 1)
    work = logits
    sel = jnp.zeros(logits.shape, dtype=jnp.bool_)
    idx_acc = jnp.zeros((t, _TOP_K), jnp.int32)
    max0 = None
    for j in range(_TOP_K):
        m = jnp.max(work, axis=-1, keepdims=True)
        if j == 0:
            max0 = m
        amax = jnp.min(jnp.where(work == m, lane, _NUM_EXPERTS),
                       axis=-1, keepdims=True)
        hit = lane == amax
        sel = jnp.logical_or(sel, hit)
        work = jnp.where(hit, -jnp.inf, work)
        idx_acc = jnp.where(cols == j, amax, idx_acc)
    e = jnp.where(sel, jnp.exp(logits - max0), 0.0)
    denom = jnp.sum(e, axis=-1, keepdims=True)
    w_out_ref[...] = e / denom
    idx_out_ref[...] = idx_acc


def kernel(input, W):
    b, s, d = input.shape
    e = W.shape[0]
    n = b * s
    x = input.reshape(n, d)
    tile = _TILE
    grid = (n // tile,)
    weights, idx = pl.pallas_call(
        _router_body,
        grid=grid,
        in_specs=[
            pl.BlockSpec((tile, d), lambda i: (i, 0)),
            pl.BlockSpec((e, d), lambda i: (0, 0)),
        ],
        out_specs=[
            pl.BlockSpec((tile, e), lambda i: (i, 0)),
            pl.BlockSpec((tile, _TOP_K), lambda i: (i, 0)),
        ],
        out_shape=[
            jax.ShapeDtypeStruct((n, e), jnp.float32),
            jax.ShapeDtypeStruct((n, _TOP_K), jnp.int32),
        ],
        compiler_params=pltpu.CompilerParams(
            dimension_semantics=("arbitrary",),
        ),
    )(x, W)
    return weights.reshape(b, s, e), idx.reshape(b, s, _TOP_K)


# speedup vs baseline: 1.3814x; 1.1509x over previous
import functools

import jax
import jax.numpy as jnp
from jax.experimental import pallas as pl
from jax.experimental.pallas import tpu as pltpu

_NUM_EXPERTS = 64
_TOP_K = 8
_TILE = 1024


def _router_body(x_ref, w_ref, w_out_ref, idx_out_ref):
    x = x_ref[...]
    w = w_ref[...]
    logits = jax.lax.dot_general(
        x, w, (((1,), (1,)), ((), ())),
        preferred_element_type=jnp.float32)
    t = logits.shape[0]
    lane = jax.lax.broadcasted_iota(jnp.int32, logits.shape, 1)
    cols = jax.lax.broadcasted_iota(jnp.int32, (t, _TOP_K), 1)
    work = logits
    sel = jnp.zeros(logits.shape, dtype=jnp.bool_)
    idx_acc = jnp.zeros((t, _TOP_K), jnp.int32)
    max0 = None
    for j in range(_TOP_K):
        m = jnp.max(work, axis=-1, keepdims=True)
        if j == 0:
            max0 = m
        amax = jnp.min(jnp.where(work == m, lane, _NUM_EXPERTS),
                       axis=-1, keepdims=True)
        hit = lane == amax
        sel = jnp.logical_or(sel, hit)
        work = jnp.where(hit, -jnp.inf, work)
        idx_acc = jnp.where(cols == j, amax, idx_acc)
    e = jnp.where(sel, jnp.exp(logits - max0), 0.0)
    denom = jnp.sum(e, axis=-1, keepdims=True)
    w_out_ref[...] = e / denom
    idx_out_ref[...] = idx_acc


def kernel(input, W):
    b, s, d = input.shape
    e = W.shape[0]
    n = b * s
    x = input.reshape(n, d)
    tile = _TILE
    grid = (n // tile,)
    weights, idx = pl.pallas_call(
        _router_body,
        grid=grid,
        in_specs=[
            pl.BlockSpec((tile, d), lambda i: (i, 0)),
            pl.BlockSpec((e, d), lambda i: (0, 0)),
        ],
        out_specs=[
            pl.BlockSpec((tile, e), lambda i: (i, 0)),
            pl.BlockSpec((tile, _TOP_K), lambda i: (i, 0)),
        ],
        out_shape=[
            jax.ShapeDtypeStruct((n, e), jnp.float32),
            jax.ShapeDtypeStruct((n, _TOP_K), jnp.int32),
        ],
        compiler_params=pltpu.CompilerParams(
            dimension_semantics=("arbitrary",),
        ),
    )(x, W)
    return weights.reshape(b, s, e), idx.reshape(b, s, _TOP_K)


# speedup vs baseline: 1.5791x; 1.1432x over previous
import functools

import jax
import jax.numpy as jnp
from jax.experimental import pallas as pl
from jax.experimental.pallas import tpu as pltpu

_NUM_EXPERTS = 64
_TOP_K = 8
_TILE = 1024


def _router_body(x_ref, w_ref, w_out_ref, idx_out_ref):
    x = x_ref[...]
    w = w_ref[...]
    logits = jax.lax.dot_general(
        x, w, (((1,), (1,)), ((), ())),
        preferred_element_type=jnp.float32)
    lt = logits.T
    t = lt.shape[1]
    row = jax.lax.broadcasted_iota(jnp.int32, lt.shape, 0)
    row8 = jax.lax.broadcasted_iota(jnp.int32, (_TOP_K, t), 0)
    work = lt
    sel = jnp.zeros(lt.shape, dtype=jnp.bool_)
    idx_t = jnp.zeros((_TOP_K, t), jnp.int32)
    max0 = None
    for j in range(_TOP_K):
        m = jnp.max(work, axis=0, keepdims=True)
        if j == 0:
            max0 = m
        amax = jnp.min(jnp.where(work == m, row, _NUM_EXPERTS),
                       axis=0, keepdims=True)
        hit = row == amax
        sel = jnp.logical_or(sel, hit)
        work = jnp.where(hit, -jnp.inf, work)
        idx_t = jnp.where(row8 == j, amax, idx_t)
    e = jnp.where(sel, jnp.exp(lt - max0), 0.0)
    denom = jnp.sum(e, axis=0, keepdims=True)
    w_out_ref[...] = (e / denom).T
    idx_out_ref[...] = idx_t.T


def kernel(input, W):
    b, s, d = input.shape
    e = W.shape[0]
    n = b * s
    x = input.reshape(n, d)
    tile = _TILE
    grid = (n // tile,)
    weights, idx = pl.pallas_call(
        _router_body,
        grid=grid,
        in_specs=[
            pl.BlockSpec((tile, d), lambda i: (i, 0)),
            pl.BlockSpec((e, d), lambda i: (0, 0)),
        ],
        out_specs=[
            pl.BlockSpec((tile, e), lambda i: (i, 0)),
            pl.BlockSpec((tile, _TOP_K), lambda i: (i, 0)),
        ],
        out_shape=[
            jax.ShapeDtypeStruct((n, e), jnp.float32),
            jax.ShapeDtypeStruct((n, _TOP_K), jnp.int32),
        ],
        compiler_params=pltpu.CompilerParams(
            dimension_semantics=("arbitrary",),
        ),
    )(x, W)
    return weights.reshape(b, s, e), idx.reshape(b, s, _TOP_K)


# speedup vs baseline: 1.6288x; 1.0315x over previous
import functools

import jax
import jax.numpy as jnp
from jax.experimental import pallas as pl
from jax.experimental.pallas import tpu as pltpu

_NUM_EXPERTS = 64
_TOP_K = 8
_TILE = 1024


def _router_body(x_ref, w_ref, w_out_ref, idx_out_ref):
    x = x_ref[0]
    w = w_ref[...]
    logits = jax.lax.dot_general(
        x, w, (((1,), (1,)), ((), ())),
        preferred_element_type=jnp.float32)
    lt = logits.T
    t = lt.shape[1]
    row = jax.lax.broadcasted_iota(jnp.int32, lt.shape, 0)
    row8 = jax.lax.broadcasted_iota(jnp.int32, (_TOP_K, t), 0)
    work = lt
    sel = jnp.zeros(lt.shape, dtype=jnp.bool_)
    idx_t = jnp.zeros((_TOP_K, t), jnp.int32)
    max0 = None
    for j in range(_TOP_K):
        m = jnp.max(work, axis=0, keepdims=True)
        if j == 0:
            max0 = m
        amax = jnp.min(jnp.where(work == m, row, _NUM_EXPERTS),
                       axis=0, keepdims=True)
        hit = row == amax
        sel = jnp.logical_or(sel, hit)
        work = jnp.where(hit, -jnp.inf, work)
        idx_t = jnp.where(row8 == j, amax, idx_t)
    e = jnp.where(sel, jnp.exp(lt - max0), 0.0)
    denom = jnp.sum(e, axis=0, keepdims=True)
    w_out_ref[0] = (e / denom).T
    idx_out_ref[0] = idx_t.T


def kernel(input, W):
    b, s, d = input.shape
    e = W.shape[0]
    tile = _TILE
    grid = (b, s // tile)
    weights, idx = pl.pallas_call(
        _router_body,
        grid=grid,
        in_specs=[
            pl.BlockSpec((1, tile, d), lambda i, j: (i, j, 0)),
            pl.BlockSpec((e, d), lambda i, j: (0, 0)),
        ],
        out_specs=[
            pl.BlockSpec((1, tile, e), lambda i, j: (i, j, 0)),
            pl.BlockSpec((1, tile, _TOP_K), lambda i, j: (i, j, 0)),
        ],
        out_shape=[
            jax.ShapeDtypeStruct((b, s, e), jnp.float32),
            jax.ShapeDtypeStruct((b, s, _TOP_K), jnp.int32),
        ],
        compiler_params=pltpu.CompilerParams(
            dimension_semantics=("arbitrary", "arbitrary"),
        ),
    )(input, W)
    return weights, idx
